# Initial kernel scaffold; baseline (speedup 1.0000x reference)
import jax
import jax.numpy as jnp
from jax.experimental import pallas as pl


def kernel(a_token, m_token, a_pe, a2m_pe, Wq, Wk, Wv, Wp1, Wca, Wp2, a_relation, a2m_relation):
    raise NotImplementedError("write your pallas kernel here")


# speedup vs baseline: 492.5082x; 492.5082x over previous
import jax
import jax.numpy as jnp
from jax.experimental import pallas as pl
from jax.experimental.pallas import tpu as pltpu

BS, A, M, T, C, H = 16, 10, 320, 90, 128, 8
B = BS * T
DH = C // H
K1, K2 = 32, 3
NB = 4
F32 = jnp.float32

_INTERPRET = False


def _body(src_ref, tgt_ref, peT1_ref, rel1_ref, peT2_ref, rel2_ref,
          wq_ref, wkv_ref, wp1_ref, wca_ref, wp2_ref, out_ref):
    row80 = jax.lax.broadcasted_iota(jnp.int32, (A * H, C), 0)
    lane128 = jax.lax.broadcasted_iota(jnp.int32, (A * H, C), 1)
    mask_q = (lane128 // DH == row80 % H).astype(F32)
    rowR = jax.lax.broadcasted_iota(jnp.int32, (A * H, A * K1), 0)
    laneR = jax.lax.broadcasted_iota(jnp.int32, (A * H, A * K1), 1)
    mask_r1 = (laneR // K1 == rowR // H).astype(F32)
    rowR2 = jax.lax.broadcasted_iota(jnp.int32, (A * H, A * K2), 0)
    laneR2 = jax.lax.broadcasted_iota(jnp.int32, (A * H, A * K2), 1)
    mask_r2 = (laneR2 // K2 == rowR2 // H).astype(F32)
    ik1 = jax.lax.broadcasted_iota(jnp.int32, (A, K1, M), 1)
    ik2 = jax.lax.broadcasted_iota(jnp.int32, (A, K2, A), 1)

    nt = (((1,), (1,)), ((), ()))
    scale = 1.0 / (DH ** 0.5)

    src = src_ref[...].reshape(NB * A, C)
    q = jnp.dot(src, wq_ref[...], preferred_element_type=F32)
    tgt = tgt_ref[...].reshape(NB * M, C)
    kv = jnp.dot(tgt, wkv_ref[...], preferred_element_type=F32)

    def attend(qb, kb, vb, peTb, relb, mask_r, kk, s):
        qbd = jnp.broadcast_to(qb[:, None, :], (A, H, C)).reshape(A * H, C)
        qbd = qbd * mask_q
        sc = jax.lax.dot_general(qbd, kb, nt, preferred_element_type=F32)
        sc = sc * scale
        oh = (relb[:, None, :] == (ik1 if kk == K1 else ik2)).astype(F32)
        oh = oh.reshape(A * kk, s)
        rbd = jnp.broadcast_to(peTb[None], (A, H, A * kk)).reshape(A * H, A * kk)
        rbd = rbd * mask_r
        sc = sc + jnp.dot(rbd, oh, preferred_element_type=F32)
        mx = jnp.max(sc, axis=-1, keepdims=True)
        e = jnp.exp(sc - mx)
        p = e / jnp.sum(e, axis=-1, keepdims=True)
        o = jnp.dot(p, vb, preferred_element_type=F32)
        return (o * mask_q).reshape(A, H, C).sum(axis=1)

    ys = []
    for b in range(NB):
        yb = attend(q[b * A:(b + 1) * A],
                    kv[b * M:(b + 1) * M, :C],
                    kv[b * M:(b + 1) * M, C:],
                    peT1_ref[b], rel1_ref[b], mask_r1, K1, M)
        ys.append(yb)
    y = jnp.concatenate(ys, axis=0)
    y = jnp.dot(y, wp1_ref[...], preferred_element_type=F32)
    qkv = jnp.dot(y, wca_ref[...], preferred_element_type=F32)

    zs = []
    for b in range(NB):
        zb = attend(qkv[b * A:(b + 1) * A, :C],
                    qkv[b * A:(b + 1) * A, C:2 * C],
                    qkv[b * A:(b + 1) * A, 2 * C:],
                    peT2_ref[b], rel2_ref[b], mask_r2, K2, A)
        zs.append(zb)
    z = jnp.concatenate(zs, axis=0)
    out = jnp.dot(z, wp2_ref[...], preferred_element_type=F32)
    out_ref[...] = out.reshape(NB, A, C)


def kernel(a_token, m_token, a_pe, a2m_pe, Wq, Wk, Wv, Wp1, Wca, Wp2,
           a_relation, a2m_relation):
    src = a_token.reshape(B, A, C)
    tgt = m_token.reshape(B, M, C)
    peT1 = jnp.transpose(a2m_pe, (0, 3, 1, 2)).reshape(B, H, A * K1)
    peT2 = jnp.transpose(a_pe, (0, 3, 1, 2)).reshape(B, H, A * K2)
    wkv = jnp.concatenate([Wk, Wv], axis=1)

    grid = (B // NB,)
    bs = pl.BlockSpec
    out = pl.pallas_call(
        _body,
        grid=grid,
        in_specs=[
            bs((NB, A, C), lambda i: (i, 0, 0)),
            bs((NB, M, C), lambda i: (i, 0, 0)),
            bs((NB, H, A * K1), lambda i: (i, 0, 0)),
            bs((NB, A, M), lambda i: (i, 0, 0)),
            bs((NB, H, A * K2), lambda i: (i, 0, 0)),
            bs((NB, A, A), lambda i: (i, 0, 0)),
            bs((C, C), lambda i: (0, 0)),
            bs((C, 2 * C), lambda i: (0, 0)),
            bs((C, C), lambda i: (0, 0)),
            bs((C, 3 * C), lambda i: (0, 0)),
            bs((C, C), lambda i: (0, 0)),
        ],
        out_specs=bs((NB, A, C), lambda i: (i, 0, 0)),
        out_shape=jax.ShapeDtypeStruct((B, A, C), F32),
        compiler_params=pltpu.CompilerParams(
            dimension_semantics=("parallel",)),
        interpret=_INTERPRET,
    )(src, tgt, peT1, a2m_relation, peT2, a_relation,
      Wq, wkv, Wp1, Wca, Wp2)
    return out


# speedup vs baseline: 513.0398x; 1.0417x over previous
import jax
import jax.numpy as jnp
from jax.experimental import pallas as pl
from jax.experimental.pallas import tpu as pltpu

BS, A, M, T, C, H = 16, 10, 320, 90, 128, 8
B = BS * T
DH = C // H
K1, K2 = 32, 3
NB = 8
F32 = jnp.float32
BF16 = jnp.bfloat16

_INTERPRET = False


def _iota2(shape, dim):
    return jax.lax.broadcasted_iota(jnp.int32, shape, dim)


def _body(src_ref, tgt_ref, peT1_ref, rel1_ref, peT2_ref, rel2_ref,
          wq_ref, wkv_ref, wp1_ref, wca_ref, wp2_ref, out_ref):
    R = NB * A * H
    nt = (((1,), (1,)), ((), ()))
    scale = 1.0 / (DH ** 0.5)

    mask_q = (_iota2((R, C), 1) // DH == _iota2((R, C), 0) % H).astype(BF16)
    mask_r1 = (_iota2((R, A * K1), 1) // K1
               == _iota2((R, A * K1), 0) // H % A).astype(BF16)
    mask_r2 = (_iota2((R, A * K2), 1) // K2
               == _iota2((R, A * K2), 0) // H % A).astype(BF16)

    def expand_rows(x):
        return jnp.broadcast_to(x[:, None, :], (NB * A, H, x.shape[1])
                                ).reshape(NB * A * H, x.shape[1])

    def expand_pe(peT, kk):
        return jnp.broadcast_to(peT[:, None], (NB, A, H, A * kk)
                                ).reshape(R, A * kk)

    def one_hot(rel, kk):
        s = rel.shape[2]
        oh = (rel[:, :, None, :] == _iota2((NB, A, kk, s), 2))
        return oh.astype(BF16).reshape(NB * A * kk, s)

    src = src_ref[...].reshape(NB * A, C).astype(BF16)
    q = jnp.dot(src, wq_ref[...].astype(BF16),
                preferred_element_type=F32)
    qbd = expand_rows((q * scale).astype(BF16)) * mask_q
    tgt = tgt_ref[...].reshape(NB * M, C).astype(BF16)
    kv = jnp.dot(tgt, wkv_ref[...].astype(BF16),
                 preferred_element_type=F32).astype(BF16)
    rbd1 = expand_pe(peT1_ref[...].astype(BF16), K1) * mask_r1
    oh1 = one_hot(rel1_ref[...], K1)

    def attend(qbd_b, kb, vb, rbd_b, oh_b):
        sc = jax.lax.dot_general(qbd_b, kb, nt, preferred_element_type=F32)
        sc = sc + jnp.dot(rbd_b, oh_b, preferred_element_type=F32)
        mx = jnp.max(sc, axis=-1, keepdims=True)
        e = jnp.exp(sc - mx)
        p = (e / jnp.sum(e, axis=-1, keepdims=True)).astype(BF16)
        return jnp.dot(p, vb, preferred_element_type=F32)

    os = []
    for b in range(NB):
        os.append(attend(qbd[b * A * H:(b + 1) * A * H],
                         kv[b * M:(b + 1) * M, :C],
                         kv[b * M:(b + 1) * M, C:],
                         rbd1[b * A * H:(b + 1) * A * H],
                         oh1[b * A * K1:(b + 1) * A * K1]))
    o = jnp.concatenate(os, axis=0)
    y = (o * mask_q.astype(F32)).reshape(NB * A, H, C).sum(axis=1)
    y = jnp.dot(y.astype(BF16), wp1_ref[...].astype(BF16),
                preferred_element_type=F32)
    qkv = jnp.dot(y.astype(BF16), wca_ref[...].astype(BF16),
                  preferred_element_type=F32)

    q2bd = expand_rows((qkv[:, :C] * scale).astype(BF16)) * mask_q
    k2 = qkv[:, C:2 * C].astype(BF16)
    v2 = qkv[:, 2 * C:].astype(BF16)
    rbd2 = expand_pe(peT2_ref[...].astype(BF16), K2) * mask_r2
    oh2 = one_hot(rel2_ref[...], K2)

    os2 = []
    for b in range(NB):
        os2.append(attend(q2bd[b * A * H:(b + 1) * A * H],
                          k2[b * A:(b + 1) * A],
                          v2[b * A:(b + 1) * A],
                          rbd2[b * A * H:(b + 1) * A * H],
                          oh2[b * A * K2:(b + 1) * A * K2]))
    o2 = jnp.concatenate(os2, axis=0)
    z = (o2 * mask_q.astype(F32)).reshape(NB * A, H, C).sum(axis=1)
    out = jnp.dot(z.astype(BF16), wp2_ref[...].astype(BF16),
                  preferred_element_type=F32)
    out_ref[...] = out.reshape(NB, A, C)


def kernel(a_token, m_token, a_pe, a2m_pe, Wq, Wk, Wv, Wp1, Wca, Wp2,
           a_relation, a2m_relation):
    src = a_token.reshape(B, A, C)
    tgt = m_token.reshape(B, M, C)
    peT1 = jnp.transpose(a2m_pe, (0, 3, 1, 2)).reshape(B, H, A * K1)
    peT2 = jnp.transpose(a_pe, (0, 3, 1, 2)).reshape(B, H, A * K2)
    wkv = jnp.concatenate([Wk, Wv], axis=1)

    grid = (B // NB,)
    bs = pl.BlockSpec
    out = pl.pallas_call(
        _body,
        grid=grid,
        in_specs=[
            bs((NB, A, C), lambda i: (i, 0, 0)),
            bs((NB, M, C), lambda i: (i, 0, 0)),
            bs((NB, H, A * K1), lambda i: (i, 0, 0)),
            bs((NB, A, M), lambda i: (i, 0, 0)),
            bs((NB, H, A * K2), lambda i: (i, 0, 0)),
            bs((NB, A, A), lambda i: (i, 0, 0)),
            bs((C, C), lambda i: (0, 0)),
            bs((C, 2 * C), lambda i: (0, 0)),
            bs((C, C), lambda i: (0, 0)),
            bs((C, 3 * C), lambda i: (0, 0)),
            bs((C, C), lambda i: (0, 0)),
        ],
        out_specs=bs((NB, A, C), lambda i: (i, 0, 0)),
        out_shape=jax.ShapeDtypeStruct((B, A, C), F32),
        compiler_params=pltpu.CompilerParams(
            dimension_semantics=("parallel",)),
        interpret=_INTERPRET,
    )(src, tgt, peT1, a2m_relation, peT2, a_relation,
      Wq, wkv, Wp1, Wca, Wp2)
    return out


# speedup vs baseline: 615.8974x; 1.2005x over previous
import jax
import jax.numpy as jnp
from jax.experimental import pallas as pl
from jax.experimental.pallas import tpu as pltpu

BS, A, M, T, C, H = 16, 10, 320, 90, 128, 8
B = BS * T
DH = C // H
K1, K2 = 32, 3
NB = 8
R = NB * A * H
F32 = jnp.float32
BF16 = jnp.bfloat16

_INTERPRET = False


def _body(src_ref, tgt_ref, t1_ref, rel1_ref, t2_ref, rel2_ref,
          mq_ref, mqf_ref, ex_ref,
          wq_ref, wkv_ref, wp1_ref, wca_ref, wp2_ref, out_ref):
    nt = (((1,), (1,)), ((), ()))
    scale = 1.0 / (DH ** 0.5)
    mq = mq_ref[...]
    mqf = mqf_ref[...]
    ex = ex_ref[...]

    def tile_rows(x):
        s = x.shape[2]
        return jnp.broadcast_to(x[:, :, None, :], (NB, A, H, s)).reshape(R, s)

    src = src_ref[...].reshape(NB * A, C).astype(BF16)
    q = jnp.dot(src, wq_ref[...].astype(BF16),
                preferred_element_type=F32)
    qbd = jnp.dot(ex, (q * scale).astype(BF16),
                  preferred_element_type=F32).astype(BF16) * mq
    tgt = tgt_ref[...].reshape(NB * M, C).astype(BF16)
    kv = jnp.dot(tgt, wkv_ref[...].astype(BF16),
                 preferred_element_type=F32).astype(BF16)
    pe1 = jnp.take_along_axis(t1_ref[...].reshape(R, K1),
                              tile_rows(rel1_ref[...]), axis=1)

    def attend(qbd_b, kb, vb, pe_b):
        sc = jax.lax.dot_general(qbd_b, kb, nt, preferred_element_type=F32)
        sc = sc + pe_b
        mx = jnp.max(sc, axis=-1, keepdims=True)
        e = jnp.exp(sc - mx)
        p = (e / jnp.sum(e, axis=-1, keepdims=True)).astype(BF16)
        return jnp.dot(p, vb, preferred_element_type=F32)

    AH = A * H
    os = []
    for b in range(NB):
        os.append(attend(qbd[b * AH:(b + 1) * AH],
                         kv[b * M:(b + 1) * M, :C],
                         kv[b * M:(b + 1) * M, C:],
                         pe1[b * AH:(b + 1) * AH]))
    o = jnp.concatenate(os, axis=0)
    y = (o * mqf).reshape(NB * A, H, C).sum(axis=1)
    y = jnp.dot(y.astype(BF16), wp1_ref[...].astype(BF16),
                preferred_element_type=F32)
    qkv = jnp.dot(y.astype(BF16), wca_ref[...].astype(BF16),
                  preferred_element_type=F32)

    q2bd = jnp.dot(ex, (qkv[:, :C] * scale).astype(BF16),
                   preferred_element_type=F32).astype(BF16) * mq
    k2 = qkv[:, C:2 * C].astype(BF16)
    v2 = qkv[:, 2 * C:].astype(BF16)
    pe2 = jnp.take_along_axis(t2_ref[...].reshape(R, K2),
                              tile_rows(rel2_ref[...]), axis=1)

    os2 = []
    for b in range(NB):
        os2.append(attend(q2bd[b * AH:(b + 1) * AH],
                          k2[b * A:(b + 1) * A],
                          v2[b * A:(b + 1) * A],
                          pe2[b * AH:(b + 1) * AH]))
    o2 = jnp.concatenate(os2, axis=0)
    z = (o2 * mqf).reshape(NB * A, H, C).sum(axis=1)
    out = jnp.dot(z.astype(BF16), wp2_ref[...].astype(BF16),
                  preferred_element_type=F32)
    out_ref[...] = out.reshape(NB, A, C)


def kernel(a_token, m_token, a_pe, a2m_pe, Wq, Wk, Wv, Wp1, Wca, Wp2,
           a_relation, a2m_relation):
    src = a_token.reshape(B, A, C)
    tgt = m_token.reshape(B, M, C)
    t1 = jnp.transpose(a2m_pe, (0, 1, 3, 2)).reshape(B, A * H, K1)
    t2 = jnp.transpose(a_pe, (0, 1, 3, 2)).reshape(B, A * H, K2)
    wkv = jnp.concatenate([Wk, Wv], axis=1)

    rows = jnp.arange(R, dtype=jnp.int32)
    lanes = jnp.arange(C, dtype=jnp.int32)
    mqf = (lanes[None, :] // DH == rows[:, None] % H).astype(F32)
    mq = mqf.astype(BF16)
    ex = (rows[:, None] // H
          == jnp.arange(NB * A, dtype=jnp.int32)[None, :]).astype(BF16)

    grid = (B // NB,)
    bs = pl.BlockSpec
    out = pl.pallas_call(
        _body,
        grid=grid,
        in_specs=[
            bs((NB, A, C), lambda i: (i, 0, 0)),
            bs((NB, M, C), lambda i: (i, 0, 0)),
            bs((NB, A * H, K1), lambda i: (i, 0, 0)),
            bs((NB, A, M), lambda i: (i, 0, 0)),
            bs((NB, A * H, K2), lambda i: (i, 0, 0)),
            bs((NB, A, A), lambda i: (i, 0, 0)),
            bs((R, C), lambda i: (0, 0)),
            bs((R, C), lambda i: (0, 0)),
            bs((R, NB * A), lambda i: (0, 0)),
            bs((C, C), lambda i: (0, 0)),
            bs((C, 2 * C), lambda i: (0, 0)),
            bs((C, C), lambda i: (0, 0)),
            bs((C, 3 * C), lambda i: (0, 0)),
            bs((C, C), lambda i: (0, 0)),
        ],
        out_specs=bs((NB, A, C), lambda i: (i, 0, 0)),
        out_shape=jax.ShapeDtypeStruct((B, A, C), F32),
        compiler_params=pltpu.CompilerParams(
            dimension_semantics=("parallel",)),
        interpret=_INTERPRET,
    )(src, tgt, t1, a2m_relation, t2, a_relation, mq, mqf, ex,
      Wq, wkv, Wp1, Wca, Wp2)
    return out


# speedup vs baseline: 739.6178x; 1.2009x over previous
import jax
import jax.numpy as jnp
from jax.experimental import pallas as pl
from jax.experimental.pallas import tpu as pltpu

BS, A, M, T, C, H = 16, 10, 320, 90, 128, 8
B = BS * T
DH = C // H
K1, K2 = 32, 3
NB = 8
R = NB * A * H
F32 = jnp.float32
BF16 = jnp.bfloat16

_INTERPRET = False


def _body(src_ref, tgt_ref, pe1_ref, rel1_ref, pe2_ref, rel2_ref,
          mq_ref, mqf_ref, ex_ref, ext_ref,
          wq_ref, wkv_ref, wp1_ref, wca_ref, wp2_ref, out_ref):
    nt = (((1,), (1,)), ((), ()))
    mq = mq_ref[...]
    mqf = mqf_ref[...]
    ex = ex_ref[...]
    ext = ext_ref[...]

    def tile_rows(x):
        s = x.shape[2]
        return jnp.broadcast_to(x[:, :, None, :], (NB, A, H, s)).reshape(R, s)

    src = src_ref[...].reshape(NB * A, C).astype(BF16)
    q = jnp.dot(src, wq_ref[...], preferred_element_type=F32).astype(BF16)
    qbd = jnp.dot(ex, q, preferred_element_type=F32).astype(BF16) * mq
    tgt = tgt_ref[...].reshape(NB * M, C).astype(BF16)
    kv = jnp.dot(tgt, wkv_ref[...], preferred_element_type=F32).astype(BF16)
    t1 = jnp.transpose(pe1_ref[...], (0, 1, 3, 2)).reshape(R, K1)
    pe1 = jnp.take_along_axis(t1, tile_rows(rel1_ref[...]), axis=1)

    def attend(qbd_b, kb, vb, pe_b, mqf_b):
        sc = jax.lax.dot_general(qbd_b, kb, nt, preferred_element_type=F32)
        e = jnp.exp(sc + pe_b)
        recip = 1.0 / jnp.sum(e, axis=-1, keepdims=True)
        o = jnp.dot(e.astype(BF16), vb, preferred_element_type=F32)
        return ((o * recip) * mqf_b).astype(BF16)

    AH = A * H
    os = []
    for b in range(NB):
        os.append(attend(qbd[b * AH:(b + 1) * AH],
                         kv[b * M:(b + 1) * M, :C],
                         kv[b * M:(b + 1) * M, C:],
                         pe1[b * AH:(b + 1) * AH],
                         mqf[b * AH:(b + 1) * AH]))
    om = jnp.concatenate(os, axis=0)
    y = jnp.dot(ext, om, preferred_element_type=F32).astype(BF16)
    y = jnp.dot(y, wp1_ref[...], preferred_element_type=F32).astype(BF16)
    qkv = jnp.dot(y, wca_ref[...], preferred_element_type=F32).astype(BF16)

    q2bd = jnp.dot(ex, qkv[:, :C], preferred_element_type=F32).astype(BF16) * mq
    k2 = qkv[:, C:2 * C]
    v2 = qkv[:, 2 * C:]
    t2 = jnp.transpose(pe2_ref[...], (0, 1, 3, 2)).reshape(R, K2)
    pe2 = jnp.take_along_axis(t2, tile_rows(rel2_ref[...]), axis=1)

    os2 = []
    for b in range(NB):
        os2.append(attend(q2bd[b * AH:(b + 1) * AH],
                          k2[b * A:(b + 1) * A],
                          v2[b * A:(b + 1) * A],
                          pe2[b * AH:(b + 1) * AH],
                          mqf[b * AH:(b + 1) * AH]))
    om2 = jnp.concatenate(os2, axis=0)
    z = jnp.dot(ext, om2, preferred_element_type=F32).astype(BF16)
    out = jnp.dot(z, wp2_ref[...], preferred_element_type=F32)
    out_ref[...] = out.reshape(NB, A, C)


def kernel(a_token, m_token, a_pe, a2m_pe, Wq, Wk, Wv, Wp1, Wca, Wp2,
           a_relation, a2m_relation):
    scale = 1.0 / (DH ** 0.5)
    src = a_token.reshape(B, A, C)
    tgt = m_token.reshape(B, M, C)
    wq = (Wq * scale).astype(BF16)
    wkv = jnp.concatenate([Wk, Wv], axis=1).astype(BF16)
    wca = jnp.concatenate([Wca[:, :C] * scale, Wca[:, C:]],
                          axis=1).astype(BF16)
    wp1 = Wp1.astype(BF16)
    wp2 = Wp2.astype(BF16)

    rows = jnp.arange(R, dtype=jnp.int32)
    lanes = jnp.arange(C, dtype=jnp.int32)
    mqf = (lanes[None, :] // DH == rows[:, None] % H).astype(F32)
    mq = mqf.astype(BF16)
    ba = jnp.arange(NB * A, dtype=jnp.int32)
    ex = (rows[:, None] // H == ba[None, :]).astype(BF16)
    ext = (ba[:, None] == rows[None, :] // H).astype(BF16)

    grid = (B // NB,)
    bs = pl.BlockSpec
    out = pl.pallas_call(
        _body,
        grid=grid,
        in_specs=[
            bs((NB, A, C), lambda i: (i, 0, 0)),
            bs((NB, M, C), lambda i: (i, 0, 0)),
            bs((NB, A, K1, H), lambda i: (i, 0, 0, 0)),
            bs((NB, A, M), lambda i: (i, 0, 0)),
            bs((NB, A, K2, H), lambda i: (i, 0, 0, 0)),
            bs((NB, A, A), lambda i: (i, 0, 0)),
            bs((R, C), lambda i: (0, 0)),
            bs((R, C), lambda i: (0, 0)),
            bs((R, NB * A), lambda i: (0, 0)),
            bs((NB * A, R), lambda i: (0, 0)),
            bs((C, C), lambda i: (0, 0)),
            bs((C, 2 * C), lambda i: (0, 0)),
            bs((C, C), lambda i: (0, 0)),
            bs((C, 3 * C), lambda i: (0, 0)),
            bs((C, C), lambda i: (0, 0)),
        ],
        out_specs=bs((NB, A, C), lambda i: (i, 0, 0)),
        out_shape=jax.ShapeDtypeStruct((B, A, C), F32),
        compiler_params=pltpu.CompilerParams(
            dimension_semantics=("parallel",)),
        interpret=_INTERPRET,
    )(src, tgt, a2m_pe, a2m_relation, a_pe, a_relation,
      mq, mqf, ex, ext, wq, wkv, wp1, wca, wp2)
    return out
